# Initial kernel scaffold; baseline (speedup 1.0000x reference)
import jax
import jax.numpy as jnp
from jax.experimental import pallas as pl


def kernel(features, memory, fitness):
    raise NotImplementedError("write your pallas kernel here")


# speedup vs baseline: 4.4063x; 4.4063x over previous
import jax
import jax.numpy as jnp
from jax.experimental import pallas as pl
from jax.experimental.pallas import tpu as pltpu

_BLOCK = 4096


def _emb_write_body(nf, feat_ref, mem_ref, fit_ref, out_mem_ref, out_fit_ref):
    i = pl.program_id(0)

    @pl.when(i < nf)
    def _():
        out_mem_ref[...] = feat_ref[...]
        out_fit_ref[...] = jnp.ones_like(out_fit_ref)

    @pl.when(i >= nf)
    def _():
        out_mem_ref[...] = mem_ref[...]
        out_fit_ref[...] = fit_ref[...]


def kernel(features, memory, fitness):
    B = features.shape[0]
    cap, dim = memory.shape
    block = _BLOCK if B % _BLOCK == 0 else 2048
    nf = B // block
    grid = (cap + block - 1) // block

    def feat_map(i):
        return (jnp.minimum(i, nf - 1), 0)

    def mem_map(i):
        return (jnp.maximum(i, nf), 0)

    def fit_map(i):
        return (jnp.maximum(i, nf),)

    import functools
    out_mem, out_fit = pl.pallas_call(
        functools.partial(_emb_write_body, nf),
        grid=(grid,),
        out_shape=(
            jax.ShapeDtypeStruct((cap, dim), memory.dtype),
            jax.ShapeDtypeStruct((cap,), fitness.dtype),
        ),
        in_specs=[
            pl.BlockSpec((block, dim), feat_map),
            pl.BlockSpec((block, dim), mem_map),
            pl.BlockSpec((block,), fit_map),
        ],
        out_specs=(
            pl.BlockSpec((block, dim), lambda i: (i, 0)),
            pl.BlockSpec((block,), lambda i: (i,)),
        ),
    )(features, memory, fitness)
    return out_mem, out_fit


# speedup vs baseline: 4.9439x; 1.1220x over previous
import jax
import jax.numpy as jnp
from jax.experimental import pallas as pl
from jax.experimental.pallas import tpu as pltpu

_BLOCK = 8192


def _emb_write_body(nf, feat_ref, mem_ref, fit_ref, out_mem_ref, out_fit_ref):
    i = pl.program_id(0)

    @pl.when(i < nf)
    def _():
        out_mem_ref[...] = feat_ref[...]
        out_fit_ref[...] = jnp.ones_like(out_fit_ref)

    @pl.when(i >= nf)
    def _():
        out_mem_ref[...] = mem_ref[...]
        out_fit_ref[...] = fit_ref[...]


def kernel(features, memory, fitness):
    B = features.shape[0]
    cap, dim = memory.shape
    block = _BLOCK if B % _BLOCK == 0 else 2048
    nf = B // block
    grid = (cap + block - 1) // block

    def feat_map(i):
        return (jnp.minimum(i, nf - 1), 0)

    def mem_map(i):
        return (jnp.maximum(i, nf), 0)

    def fit_map(i):
        return (jnp.maximum(i, nf),)

    import functools
    out_mem, out_fit = pl.pallas_call(
        functools.partial(_emb_write_body, nf),
        grid=(grid,),
        out_shape=(
            jax.ShapeDtypeStruct((cap, dim), memory.dtype),
            jax.ShapeDtypeStruct((cap,), fitness.dtype),
        ),
        in_specs=[
            pl.BlockSpec((block, dim), feat_map),
            pl.BlockSpec((block, dim), mem_map),
            pl.BlockSpec((block,), fit_map),
        ],
        out_specs=(
            pl.BlockSpec((block, dim), lambda i: (i, 0)),
            pl.BlockSpec((block,), lambda i: (i,)),
        ),
    )(features, memory, fitness)
    return out_mem, out_fit
